# baseline (speedup 1.0000x reference)
import jax
import jax.numpy as jnp
from jax import lax
from jax.experimental import pallas as pl
from jax.experimental.pallas import tpu as pltpu
from jax.experimental.pallas import tpu_sc as plsc

T = 4096
D = 1024
E = 16
HS = 4096
HR = 1024
K = 2
A = T * K
TM = 256
NT = A // TM + E
P = NT * TM
NC, NS, L = 2, 16, 16
NW = NC * NS


def _dotT(a, b):
    return jax.lax.dot_general(a, b, (((1,), (1,)), ((), ())),
                               preferred_element_type=jnp.float32)


def _dot_hi(a, b):
    return jax.lax.dot_general(a, b, (((1,), (0,)), ((), ())),
                               preferred_element_type=jnp.float32,
                               precision=jax.lax.Precision.HIGHEST)


def _router_body(x_ref, rw_ref, bias_ref, pos_ref, gate_ref, te_ref, used_ref):
    x = x_ref[...]
    scores = jax.nn.sigmoid(_dotT(x, rw_ref[...]))
    sel = scores + bias_ref[...]
    cols = lax.broadcasted_iota(jnp.int32, (T, E), 1)
    m1 = jnp.max(sel, axis=1, keepdims=True)
    i1 = jnp.min(jnp.where(sel == m1, cols, E), axis=1, keepdims=True)
    mask1 = cols == i1
    sel2 = jnp.where(mask1, -jnp.inf, sel)
    m2 = jnp.max(sel2, axis=1, keepdims=True)
    i2 = jnp.min(jnp.where(sel2 == m2, cols, E), axis=1, keepdims=True)
    mask2 = cols == i2
    selk = (mask1 | mask2).astype(jnp.float32)

    BB = 512
    rix = lax.broadcasted_iota(jnp.int32, (BB, BB), 0)
    cix = lax.broadcasted_iota(jnp.int32, (BB, BB), 1)
    tril = (cix < rix).astype(jnp.float32)
    carry = jnp.zeros((1, E), jnp.float32)
    rblocks = []
    for bi in range(T // BB):
        blk = selk[bi * BB:(bi + 1) * BB, :]
        rblocks.append(_dot_hi(tril, blk) + carry)
        carry = carry + jnp.sum(blk, axis=0, keepdims=True)
    ranks = jnp.concatenate(rblocks, axis=0)
    counts = carry
    padded = jnp.ceil(counts / TM) * TM

    eri = lax.broadcasted_iota(jnp.int32, (E, E), 0)
    eci = lax.broadcasted_iota(jnp.int32, (E, E), 1)
    etril = (eri < eci).astype(jnp.float32)
    seg = _dot_hi(padded, etril)

    posd = seg + ranks
    p1 = jnp.sum(jnp.where(mask1, posd, 0.0), axis=1, keepdims=True)
    p2 = jnp.sum(jnp.where(mask2, posd, 0.0), axis=1, keepdims=True)
    g1 = jnp.sum(jnp.where(mask1, scores, 0.0), axis=1, keepdims=True)
    g2 = jnp.sum(jnp.where(mask2, scores, 0.0), axis=1, keepdims=True)
    pos_ref[...] = jnp.concatenate([p1, p2], axis=1).astype(jnp.int32)
    gate_ref[...] = jnp.concatenate([g1, g2], axis=1)

    tstart = (lax.broadcasted_iota(jnp.int32, (NT, 1), 0) * TM).astype(jnp.float32)
    te_ref[...] = (jnp.sum((seg <= tstart).astype(jnp.int32), axis=1,
                           keepdims=True) - 1)
    total = jnp.sum(padded, axis=1, keepdims=True)
    used_ref[...] = (tstart < total).astype(jnp.int32)


def _sc_scatter_body(pos_hbm, gate_hbm, tok_out, gsort_out,
                     posv, gatev, tokbuf, gatebuf):
    wid = lax.axis_index("s") * NC + lax.axis_index("c")

    @pl.when(wid == 0)
    def _():
        pltpu.sync_copy(pos_hbm, posv)
        pltpu.sync_copy(gate_hbm, gatev)

        def zero_body(i, c):
            tokbuf[pl.ds(i * L, L)] = jnp.zeros((L,), jnp.int32)
            gatebuf[pl.ds(i * L, L)] = jnp.zeros((L,), jnp.float32)
            return c
        lax.fori_loop(0, P // L, zero_body, 0)

        iota = lax.iota(jnp.int32, L)

        def scat_body(j, c):
            idx = posv[pl.ds(j * L, L)]
            tok = lax.shift_right_logical(iota + j * L, 1)
            g = gatev[pl.ds(j * L, L)]
            plsc.store_scatter(tokbuf, [idx], tok)
            plsc.store_scatter(gatebuf, [idx], g)
            return c
        lax.fori_loop(0, A // L, scat_body, 0)

        pltpu.sync_copy(tokbuf, tok_out)
        pltpu.sync_copy(gatebuf, gsort_out)


RPW = P // NW
CH = 96


def _sc_gather_body(x_hbm, tok_hbm, xs_out, idxv, b0, b1, g0, g1, s0, s1):
    bufs, gsems, ssems = [b0, b1], [g0, g1], [s0, s1]
    wid = lax.axis_index("s") * NC + lax.axis_index("c")
    base = wid * RPW
    pltpu.sync_copy(tok_hbm.at[pl.ds(base, RPW)], idxv)
    nch = RPW // CH
    gp, sp = {}, {}

    def startg(c):
        k = c & 1
        gp[c] = pltpu.async_copy(x_hbm.at[idxv.at[pl.ds(c * CH, CH)]],
                                 bufs[k], gsems[k])

    startg(0)
    for c in range(nch):
        k = c & 1
        gp.pop(c).wait()
        sp[c] = pltpu.async_copy(bufs[k], xs_out.at[pl.ds(base + c * CH, CH)],
                                 ssems[k])
        if c + 1 < nch:
            if c - 1 >= 0:
                sp.pop(c - 1).wait()
            startg(c + 1)
    sp.pop(nch - 1).wait()


def _gmm_body(te_ref, used_ref, x_ref, w1_ref, w2_ref, w3_ref, g_ref, y_ref):
    i = pl.program_id(0)

    @pl.when(used_ref[i] != 0)
    def _():
        x = x_ref[...].astype(jnp.float32)
        h = jax.nn.silu(_dotT(x, w1_ref[0])) * _dotT(x, w2_ref[0])
        y_ref[...] = _dotT(h, w3_ref[0]) * g_ref[...]


def _shared_body(x_ref, w1_ref, w2_ref, w3_ref, out_ref):
    j = pl.program_id(1)
    x = x_ref[...]
    h = jax.nn.silu(_dotT(x, w1_ref[...])) * _dotT(x, w2_ref[...])
    part = _dotT(h, w3_ref[...])

    @pl.when(j == 0)
    def _():
        out_ref[...] = part

    @pl.when(j != 0)
    def _():
        out_ref[...] += part


TPW = T // NW
TCH = 16


def _sc_combine_body(y_hbm, pos_hbm, sh_hbm, out_hbm,
                     idx0, idx1, yb0, yb1, acc0, acc1, g0, g1, s0, s1):
    idxs, ybs, accs = [idx0, idx1], [yb0, yb1], [acc0, acc1]
    gsems, ssems = [g0, g1], [s0, s1]
    wid = lax.axis_index("s") * NC + lax.axis_index("c")
    tbase = wid * TPW
    nch = TPW // TCH
    pend = {}

    def start(c):
        k = c & 1
        t0 = tbase + c * TCH
        pltpu.sync_copy(pos_hbm.at[pl.ds(t0 * K, TCH * K)], idxs[k])
        pend[c] = (pltpu.async_copy(y_hbm.at[idxs[k]], ybs[k], gsems[k]),
                   pltpu.async_copy(sh_hbm.at[pl.ds(t0, TCH)], accs[k],
                                    ssems[k]))

    start(0)
    for c in range(nch):
        k = c & 1
        t0 = tbase + c * TCH
        cp_y, cp_s = pend.pop(c)
        cp_y.wait()
        cp_s.wait()
        if c + 1 < nch:
            start(c + 1)
        acc, yb = accs[k], ybs[k]
        for i in range(TCH):
            def vbody(v, c2, i=i):
                sl = pl.ds(v * L, L)
                acc[i, sl] = acc[i, sl] + yb[2 * i, sl] + yb[2 * i + 1, sl]
                return c2
            lax.fori_loop(0, D // L, vbody, 0)
        pltpu.sync_copy(acc, out_hbm.at[pl.ds(t0, TCH)])


def _sc_mesh():
    return plsc.VectorSubcoreMesh(core_axis_name="c", subcore_axis_name="s",
                                  num_cores=NC, num_subcores=NS)


def kernel(x, shared_w1, shared_w2, shared_w3, routed_w1, routed_w2, routed_w3,
           router_w, expert_bias):
    b, s, d = x.shape
    _SC_MESH = _sc_mesh()
    x2 = x.reshape(T, D)

    pos2, gate2, te2, used2 = pl.pallas_call(
        _router_body,
        out_shape=[
            jax.ShapeDtypeStruct((T, K), jnp.int32),
            jax.ShapeDtypeStruct((T, K), jnp.float32),
            jax.ShapeDtypeStruct((NT, 1), jnp.int32),
            jax.ShapeDtypeStruct((NT, 1), jnp.int32),
        ],
    )(x2, router_w, expert_bias.reshape(1, E))
    pos8 = pos2.reshape(A)
    gate8 = gate2.reshape(A)
    te = te2.reshape(NT)
    used = used2.reshape(NT)

    tok_sorted, gate_sorted = pl.kernel(
        _sc_scatter_body,
        out_type=[jax.ShapeDtypeStruct((P,), jnp.int32),
                  jax.ShapeDtypeStruct((P,), jnp.float32)],
        mesh=_SC_MESH,
        scratch_types=[pltpu.VMEM((A,), jnp.int32),
                       pltpu.VMEM((A,), jnp.float32),
                       pltpu.VMEM((P,), jnp.int32),
                       pltpu.VMEM((P,), jnp.float32)],
        compiler_params=pltpu.CompilerParams(needs_layout_passes=False),
    )(pos8, gate8)

    xb = x2.astype(jnp.bfloat16)
    xb_i32 = jax.lax.bitcast_convert_type(xb.reshape(T, D // 2, 2), jnp.int32)
    xs_i32 = pl.kernel(
        _sc_gather_body,
        out_type=jax.ShapeDtypeStruct((P, D // 2), jnp.int32),
        mesh=_SC_MESH,
        scratch_types=[pltpu.VMEM((RPW,), jnp.int32),
                       pltpu.VMEM((CH, D // 2), jnp.int32),
                       pltpu.VMEM((CH, D // 2), jnp.int32),
                       pltpu.SemaphoreType.DMA,
                       pltpu.SemaphoreType.DMA,
                       pltpu.SemaphoreType.DMA,
                       pltpu.SemaphoreType.DMA],
    )(xb_i32, tok_sorted)
    xs = jax.lax.bitcast_convert_type(xs_i32, jnp.bfloat16).reshape(P, D)

    y = pl.pallas_call(
        _gmm_body,
        grid_spec=pltpu.PrefetchScalarGridSpec(
            num_scalar_prefetch=2,
            grid=(NT,),
            in_specs=[
                pl.BlockSpec((TM, D), lambda i, te, us: (i, 0)),
                pl.BlockSpec((1, HR, D), lambda i, te, us: (te[i], 0, 0)),
                pl.BlockSpec((1, HR, D), lambda i, te, us: (te[i], 0, 0)),
                pl.BlockSpec((1, D, HR), lambda i, te, us: (te[i], 0, 0)),
                pl.BlockSpec((TM, 1), lambda i, te, us: (i, 0)),
            ],
            out_specs=pl.BlockSpec((TM, D), lambda i, te, us: (i, 0)),
        ),
        out_shape=jax.ShapeDtypeStruct((P, D), jnp.float32),
    )(te, used, xs, routed_w1, routed_w2, routed_w3,
      gate_sorted.reshape(P, 1))

    BS = 512
    HB = 1024
    sh = pl.pallas_call(
        _shared_body,
        grid=(T // BS, HS // HB),
        in_specs=[
            pl.BlockSpec((BS, D), lambda t, j: (t, 0)),
            pl.BlockSpec((HB, D), lambda t, j: (j, 0)),
            pl.BlockSpec((HB, D), lambda t, j: (j, 0)),
            pl.BlockSpec((D, HB), lambda t, j: (0, j)),
        ],
        out_specs=pl.BlockSpec((BS, D), lambda t, j: (t, 0)),
        out_shape=jax.ShapeDtypeStruct((T, D), jnp.float32),
    )(x2, shared_w1, shared_w2, shared_w3)

    out = pl.kernel(
        _sc_combine_body,
        out_type=jax.ShapeDtypeStruct((T, D), jnp.float32),
        mesh=_SC_MESH,
        scratch_types=[pltpu.VMEM((TCH * K,), jnp.int32),
                       pltpu.VMEM((TCH * K,), jnp.int32),
                       pltpu.VMEM((TCH * K, D), jnp.float32),
                       pltpu.VMEM((TCH * K, D), jnp.float32),
                       pltpu.VMEM((TCH, D), jnp.float32),
                       pltpu.VMEM((TCH, D), jnp.float32),
                       pltpu.SemaphoreType.DMA,
                       pltpu.SemaphoreType.DMA,
                       pltpu.SemaphoreType.DMA,
                       pltpu.SemaphoreType.DMA],
    )(y, pos8, sh)

    return out.reshape(b, s, d)


# speedup vs baseline: 2.0378x; 2.0378x over previous
import jax
import jax.numpy as jnp
from jax import lax
from jax.experimental import pallas as pl
from jax.experimental.pallas import tpu as pltpu
from jax.experimental.pallas import tpu_sc as plsc

T = 4096
D = 1024
E = 16
HS = 4096
HR = 1024
K = 2
A = T * K
TM = 256
NT = A // TM + E
P = NT * TM
NC, NS, L = 2, 16, 16
NW = NC * NS


def _dotT(a, b):
    return jax.lax.dot_general(a, b, (((1,), (1,)), ((), ())),
                               preferred_element_type=jnp.float32)


def _dot_hi(a, b):
    return jax.lax.dot_general(a, b, (((1,), (0,)), ((), ())),
                               preferred_element_type=jnp.float32,
                               precision=jax.lax.Precision.HIGHEST)


def _router_body(x_ref, rw_ref, bias_ref, pos_ref, gate_ref, te_ref, used_ref):
    x = x_ref[...]
    scores = jax.nn.sigmoid(_dotT(x, rw_ref[...]))
    sel = scores + bias_ref[...]
    cols = lax.broadcasted_iota(jnp.int32, (T, E), 1)
    m1 = jnp.max(sel, axis=1, keepdims=True)
    i1 = jnp.min(jnp.where(sel == m1, cols, E), axis=1, keepdims=True)
    mask1 = cols == i1
    sel2 = jnp.where(mask1, -jnp.inf, sel)
    m2 = jnp.max(sel2, axis=1, keepdims=True)
    i2 = jnp.min(jnp.where(sel2 == m2, cols, E), axis=1, keepdims=True)
    mask2 = cols == i2
    selk = (mask1 | mask2).astype(jnp.float32)

    BB = 512
    rix = lax.broadcasted_iota(jnp.int32, (BB, BB), 0)
    cix = lax.broadcasted_iota(jnp.int32, (BB, BB), 1)
    tril = (cix < rix).astype(jnp.float32)
    carry = jnp.zeros((1, E), jnp.float32)
    rblocks = []
    for bi in range(T // BB):
        blk = selk[bi * BB:(bi + 1) * BB, :]
        rblocks.append(_dot_hi(tril, blk) + carry)
        carry = carry + jnp.sum(blk, axis=0, keepdims=True)
    ranks = jnp.concatenate(rblocks, axis=0)
    counts = carry
    padded = jnp.ceil(counts / TM) * TM

    eri = lax.broadcasted_iota(jnp.int32, (E, E), 0)
    eci = lax.broadcasted_iota(jnp.int32, (E, E), 1)
    etril = (eri < eci).astype(jnp.float32)
    seg = _dot_hi(padded, etril)

    posd = seg + ranks
    p1 = jnp.sum(jnp.where(mask1, posd, 0.0), axis=1, keepdims=True)
    p2 = jnp.sum(jnp.where(mask2, posd, 0.0), axis=1, keepdims=True)
    g1 = jnp.sum(jnp.where(mask1, scores, 0.0), axis=1, keepdims=True)
    g2 = jnp.sum(jnp.where(mask2, scores, 0.0), axis=1, keepdims=True)
    pos_ref[...] = jnp.concatenate([p1, p2], axis=1).astype(jnp.int32)
    gate_ref[...] = jnp.concatenate([g1, g2], axis=1)

    tstart = (lax.broadcasted_iota(jnp.int32, (NT, 1), 0) * TM).astype(jnp.float32)
    te_ref[...] = (jnp.sum((seg <= tstart).astype(jnp.int32), axis=1,
                           keepdims=True) - 1)
    total = jnp.sum(padded, axis=1, keepdims=True)
    used_ref[...] = (tstart < total).astype(jnp.int32)


def _sc_scatter_body(pos_hbm, gate_hbm, tok_out, gsort_out,
                     posv, gatev, tokbuf, gatebuf):
    wid = lax.axis_index("s") * NC + lax.axis_index("c")

    @pl.when(wid == 0)
    def _():
        pltpu.sync_copy(pos_hbm, posv)
        pltpu.sync_copy(gate_hbm, gatev)

        def zero_body(i, c):
            tokbuf[pl.ds(i * L, L)] = jnp.zeros((L,), jnp.int32)
            gatebuf[pl.ds(i * L, L)] = jnp.zeros((L,), jnp.float32)
            return c
        lax.fori_loop(0, P // L, zero_body, 0)

        iota = lax.iota(jnp.int32, L)

        def scat_body(j, c):
            idx = posv[pl.ds(j * L, L)]
            tok = lax.shift_right_logical(iota + j * L, 1)
            g = gatev[pl.ds(j * L, L)]
            plsc.store_scatter(tokbuf, [idx], tok)
            plsc.store_scatter(gatebuf, [idx], g)
            return c
        lax.fori_loop(0, A // L, scat_body, 0)

        pltpu.sync_copy(tokbuf, tok_out)
        pltpu.sync_copy(gatebuf, gsort_out)


def _gmm_body(te_ref, used_ref, tok_ref, xb_ref, w1_ref, w2_ref, w3_ref,
              g_ref, y_ref):
    i = pl.program_id(0)

    @pl.when(used_ref[i] != 0)
    def _():
        tok = tok_ref[...]
        tcols = lax.broadcasted_iota(jnp.int32, (TM, T), 1)
        onehot = (tcols == tok).astype(jnp.bfloat16)
        xg = jax.lax.dot_general(onehot, xb_ref[...], (((1,), (0,)), ((), ())),
                                 preferred_element_type=jnp.float32)
        h = jax.nn.silu(_dotT(xg, w1_ref[0])) * _dotT(xg, w2_ref[0])
        y_ref[...] = _dotT(h, w3_ref[0]) * g_ref[...]


def _shared_body(x_ref, w1_ref, w2_ref, w3_ref, out_ref):
    j = pl.program_id(1)
    x = x_ref[...]
    h = jax.nn.silu(_dotT(x, w1_ref[...])) * _dotT(x, w2_ref[...])
    part = _dotT(h, w3_ref[...])

    @pl.when(j == 0)
    def _():
        out_ref[...] = part

    @pl.when(j != 0)
    def _():
        out_ref[...] += part


TPW = T // NW
TCH = 16


def _sc_combine_body(y_hbm, pos_hbm, sh_hbm, out_hbm,
                     idx0, idx1, yb0, yb1, acc0, acc1, g0, g1, s0, s1):
    idxs, ybs, accs = [idx0, idx1], [yb0, yb1], [acc0, acc1]
    gsems, ssems = [g0, g1], [s0, s1]
    wid = lax.axis_index("s") * NC + lax.axis_index("c")
    tbase = wid * TPW
    nch = TPW // TCH
    pend = {}

    def start(c):
        k = c & 1
        t0 = tbase + c * TCH
        pltpu.sync_copy(pos_hbm.at[pl.ds(t0 * K, TCH * K)], idxs[k])
        pend[c] = (pltpu.async_copy(y_hbm.at[idxs[k]], ybs[k], gsems[k]),
                   pltpu.async_copy(sh_hbm.at[pl.ds(t0, TCH)], accs[k],
                                    ssems[k]))

    start(0)
    for c in range(nch):
        k = c & 1
        t0 = tbase + c * TCH
        cp_y, cp_s = pend.pop(c)
        cp_y.wait()
        cp_s.wait()
        if c + 1 < nch:
            start(c + 1)
        acc, yb = accs[k], ybs[k]
        for i in range(TCH):
            def vbody(v, c2, i=i):
                sl = pl.ds(v * L, L)
                acc[i, sl] = acc[i, sl] + yb[2 * i, sl] + yb[2 * i + 1, sl]
                return c2
            lax.fori_loop(0, D // L, vbody, 0)
        pltpu.sync_copy(acc, out_hbm.at[pl.ds(t0, TCH)])


def _sc_mesh():
    return plsc.VectorSubcoreMesh(core_axis_name="c", subcore_axis_name="s",
                                  num_cores=NC, num_subcores=NS)


def kernel(x, shared_w1, shared_w2, shared_w3, routed_w1, routed_w2, routed_w3,
           router_w, expert_bias):
    b, s, d = x.shape
    _SC_MESH = _sc_mesh()
    x2 = x.reshape(T, D)

    pos2, gate2, te2, used2 = pl.pallas_call(
        _router_body,
        out_shape=[
            jax.ShapeDtypeStruct((T, K), jnp.int32),
            jax.ShapeDtypeStruct((T, K), jnp.float32),
            jax.ShapeDtypeStruct((NT, 1), jnp.int32),
            jax.ShapeDtypeStruct((NT, 1), jnp.int32),
        ],
    )(x2, router_w, expert_bias.reshape(1, E))
    pos8 = pos2.reshape(A)
    gate8 = gate2.reshape(A)
    te = te2.reshape(NT)
    used = used2.reshape(NT)

    tok_sorted, gate_sorted = pl.kernel(
        _sc_scatter_body,
        out_type=[jax.ShapeDtypeStruct((P,), jnp.int32),
                  jax.ShapeDtypeStruct((P,), jnp.float32)],
        mesh=_SC_MESH,
        scratch_types=[pltpu.VMEM((A,), jnp.int32),
                       pltpu.VMEM((A,), jnp.float32),
                       pltpu.VMEM((P,), jnp.int32),
                       pltpu.VMEM((P,), jnp.float32)],
        compiler_params=pltpu.CompilerParams(needs_layout_passes=False),
    )(pos8, gate8)

    xb = x2.astype(jnp.bfloat16)
    y = pl.pallas_call(
        _gmm_body,
        grid_spec=pltpu.PrefetchScalarGridSpec(
            num_scalar_prefetch=2,
            grid=(NT,),
            in_specs=[
                pl.BlockSpec((TM, 1), lambda i, te, us: (i, 0)),
                pl.BlockSpec((T, D), lambda i, te, us: (0, 0)),
                pl.BlockSpec((1, HR, D), lambda i, te, us: (te[i], 0, 0)),
                pl.BlockSpec((1, HR, D), lambda i, te, us: (te[i], 0, 0)),
                pl.BlockSpec((1, D, HR), lambda i, te, us: (te[i], 0, 0)),
                pl.BlockSpec((TM, 1), lambda i, te, us: (i, 0)),
            ],
            out_specs=pl.BlockSpec((TM, D), lambda i, te, us: (i, 0)),
        ),
        out_shape=jax.ShapeDtypeStruct((P, D), jnp.float32),
    )(te, used, tok_sorted.reshape(P, 1), xb, routed_w1, routed_w2, routed_w3,
      gate_sorted.reshape(P, 1))

    BS = 512
    HB = 1024
    sh = pl.pallas_call(
        _shared_body,
        grid=(T // BS, HS // HB),
        in_specs=[
            pl.BlockSpec((BS, D), lambda t, j: (t, 0)),
            pl.BlockSpec((HB, D), lambda t, j: (j, 0)),
            pl.BlockSpec((HB, D), lambda t, j: (j, 0)),
            pl.BlockSpec((D, HB), lambda t, j: (0, j)),
        ],
        out_specs=pl.BlockSpec((BS, D), lambda t, j: (t, 0)),
        out_shape=jax.ShapeDtypeStruct((T, D), jnp.float32),
    )(x2, shared_w1, shared_w2, shared_w3)

    out = pl.kernel(
        _sc_combine_body,
        out_type=jax.ShapeDtypeStruct((T, D), jnp.float32),
        mesh=_SC_MESH,
        scratch_types=[pltpu.VMEM((TCH * K,), jnp.int32),
                       pltpu.VMEM((TCH * K,), jnp.int32),
                       pltpu.VMEM((TCH * K, D), jnp.float32),
                       pltpu.VMEM((TCH * K, D), jnp.float32),
                       pltpu.VMEM((TCH, D), jnp.float32),
                       pltpu.VMEM((TCH, D), jnp.float32),
                       pltpu.SemaphoreType.DMA,
                       pltpu.SemaphoreType.DMA,
                       pltpu.SemaphoreType.DMA,
                       pltpu.SemaphoreType.DMA],
    )(y, pos8, sh)

    return out.reshape(b, s, d)


# speedup vs baseline: 2.0769x; 1.0192x over previous
import jax
import jax.numpy as jnp
from jax import lax
from jax.experimental import pallas as pl
from jax.experimental.pallas import tpu as pltpu
from jax.experimental.pallas import tpu_sc as plsc

T = 4096
D = 1024
E = 16
HS = 4096
HR = 1024
K = 2
A = T * K
TM = 256
NT = A // TM + E
P = NT * TM
NC, NS, L = 2, 16, 16
NW = NC * NS


def _dotT(a, b):
    return jax.lax.dot_general(a, b, (((1,), (1,)), ((), ())),
                               preferred_element_type=jnp.float32)


def _dot_hi(a, b):
    return jax.lax.dot_general(a, b, (((1,), (0,)), ((), ())),
                               preferred_element_type=jnp.float32,
                               precision=jax.lax.Precision.HIGHEST)


def _router_body(x_ref, rw_ref, bias_ref, pos_ref, gate_ref, te_ref, used_ref,
                 xb_ref):
    x = x_ref[...]
    xb_ref[...] = x.astype(jnp.bfloat16)
    scores = jax.nn.sigmoid(_dotT(x, rw_ref[...]))
    sel = scores + bias_ref[...]
    cols = lax.broadcasted_iota(jnp.int32, (T, E), 1)
    m1 = jnp.max(sel, axis=1, keepdims=True)
    i1 = jnp.min(jnp.where(sel == m1, cols, E), axis=1, keepdims=True)
    mask1 = cols == i1
    sel2 = jnp.where(mask1, -jnp.inf, sel)
    m2 = jnp.max(sel2, axis=1, keepdims=True)
    i2 = jnp.min(jnp.where(sel2 == m2, cols, E), axis=1, keepdims=True)
    mask2 = cols == i2
    selk = (mask1 | mask2).astype(jnp.float32)

    BB = 512
    rix = lax.broadcasted_iota(jnp.int32, (BB, BB), 0)
    cix = lax.broadcasted_iota(jnp.int32, (BB, BB), 1)
    tril = (cix < rix).astype(jnp.float32)
    carry = jnp.zeros((1, E), jnp.float32)
    rblocks = []
    for bi in range(T // BB):
        blk = selk[bi * BB:(bi + 1) * BB, :]
        rblocks.append(_dot_hi(tril, blk) + carry)
        carry = carry + jnp.sum(blk, axis=0, keepdims=True)
    ranks = jnp.concatenate(rblocks, axis=0)
    counts = carry
    padded = jnp.ceil(counts / TM) * TM

    eri = lax.broadcasted_iota(jnp.int32, (E, E), 0)
    eci = lax.broadcasted_iota(jnp.int32, (E, E), 1)
    etril = (eri < eci).astype(jnp.float32)
    seg = _dot_hi(padded, etril)

    posd = seg + ranks
    p1 = jnp.sum(jnp.where(mask1, posd, 0.0), axis=1, keepdims=True)
    p2 = jnp.sum(jnp.where(mask2, posd, 0.0), axis=1, keepdims=True)
    g1 = jnp.sum(jnp.where(mask1, scores, 0.0), axis=1, keepdims=True)
    g2 = jnp.sum(jnp.where(mask2, scores, 0.0), axis=1, keepdims=True)
    pos_ref[...] = jnp.concatenate([p1, p2], axis=1).astype(jnp.int32)
    gate_ref[...] = jnp.concatenate([g1, g2], axis=1)

    tstart = (lax.broadcasted_iota(jnp.int32, (NT, 1), 0) * TM).astype(jnp.float32)
    te_ref[...] = (jnp.sum((seg <= tstart).astype(jnp.int32), axis=1,
                           keepdims=True) - 1)
    total = jnp.sum(padded, axis=1, keepdims=True)
    used_ref[...] = (tstart < total).astype(jnp.int32)


def _sc_scatter_body(pos_hbm, gate_hbm, tok_out, gsort_out,
                     posv, gatev, tokbuf, gatebuf):
    wid = lax.axis_index("s") * NC + lax.axis_index("c")

    @pl.when(wid == 0)
    def _():
        pltpu.sync_copy(pos_hbm, posv)
        pltpu.sync_copy(gate_hbm, gatev)

        def zero_body(i, c):
            tokbuf[pl.ds(i * L, L)] = jnp.zeros((L,), jnp.int32)
            gatebuf[pl.ds(i * L, L)] = jnp.zeros((L,), jnp.float32)
            return c
        lax.fori_loop(0, P // L, zero_body, 0)

        iota = lax.iota(jnp.int32, L)

        def scat_body(j, c):
            idx = posv[pl.ds(j * L, L)]
            tok = lax.shift_right_logical(iota + j * L, 1)
            g = gatev[pl.ds(j * L, L)]
            plsc.store_scatter(tokbuf, [idx], tok)
            plsc.store_scatter(gatebuf, [idx], g)
            return c
        lax.fori_loop(0, A // L, scat_body, 0)

        pltpu.sync_copy(tokbuf, tok_out)
        pltpu.sync_copy(gatebuf, gsort_out)


def _gmm_body(te_ref, used_ref, tok_ref, xb_ref, w1_ref, w2_ref, w3_ref,
              g_ref, y_ref):
    i = pl.program_id(0)

    @pl.when(used_ref[i] != 0)
    def _():
        tok = tok_ref[...]
        tcols = lax.broadcasted_iota(jnp.int32, (TM, T), 1)
        onehot = (tcols == tok).astype(jnp.bfloat16)
        xg = jax.lax.dot_general(onehot, xb_ref[...], (((1,), (0,)), ((), ())),
                                 preferred_element_type=jnp.float32)
        h = jax.nn.silu(_dotT(xg, w1_ref[0])) * _dotT(xg, w2_ref[0])
        y_ref[...] = _dotT(h, w3_ref[0]) * g_ref[...]


def _shared_body(x_ref, w1_ref, w2_ref, w3_ref, out_ref):
    j = pl.program_id(1)
    x = x_ref[...]
    h = jax.nn.silu(_dotT(x, w1_ref[...])) * _dotT(x, w2_ref[...])
    part = _dotT(h, w3_ref[...])

    @pl.when(j == 0)
    def _():
        out_ref[...] = part

    @pl.when(j != 0)
    def _():
        out_ref[...] += part


TPW = T // NW
TCH = 16


def _sc_combine_body(y_hbm, pos_hbm, sh_hbm, out_hbm,
                     idx0, idx1, yb0, yb1, acc0, acc1, g0, g1, s0, s1):
    idxs, ybs, accs = [idx0, idx1], [yb0, yb1], [acc0, acc1]
    gsems, ssems = [g0, g1], [s0, s1]
    wid = lax.axis_index("s") * NC + lax.axis_index("c")
    tbase = wid * TPW
    nch = TPW // TCH
    pend = {}

    def start(c):
        k = c & 1
        t0 = tbase + c * TCH
        pltpu.sync_copy(pos_hbm.at[pl.ds(t0 * K, TCH * K)], idxs[k])
        pend[c] = (pltpu.async_copy(y_hbm.at[idxs[k]], ybs[k], gsems[k]),
                   pltpu.async_copy(sh_hbm.at[pl.ds(t0, TCH)], accs[k],
                                    ssems[k]))

    start(0)
    for c in range(nch):
        k = c & 1
        t0 = tbase + c * TCH
        cp_y, cp_s = pend.pop(c)
        cp_y.wait()
        cp_s.wait()
        if c + 1 < nch:
            start(c + 1)
        acc, yb = accs[k], ybs[k]
        for i in range(TCH):
            def vbody(v, c2, i=i):
                sl = pl.ds(v * L, L)
                acc[i, sl] = acc[i, sl] + yb[2 * i, sl] + yb[2 * i + 1, sl]
                return c2
            lax.fori_loop(0, D // L, vbody, 0)
        pltpu.sync_copy(acc, out_hbm.at[pl.ds(t0, TCH)])


def _sc_mesh():
    return plsc.VectorSubcoreMesh(core_axis_name="c", subcore_axis_name="s",
                                  num_cores=NC, num_subcores=NS)


def kernel(x, shared_w1, shared_w2, shared_w3, routed_w1, routed_w2, routed_w3,
           router_w, expert_bias):
    b, s, d = x.shape
    _SC_MESH = _sc_mesh()
    x2 = x.reshape(T, D)

    pos2, gate2, te2, used2, xb = pl.pallas_call(
        _router_body,
        out_shape=[
            jax.ShapeDtypeStruct((T, K), jnp.int32),
            jax.ShapeDtypeStruct((T, K), jnp.float32),
            jax.ShapeDtypeStruct((NT, 1), jnp.int32),
            jax.ShapeDtypeStruct((NT, 1), jnp.int32),
            jax.ShapeDtypeStruct((T, D), jnp.bfloat16),
        ],
    )(x2, router_w, expert_bias.reshape(1, E))
    pos8 = pos2.reshape(A)
    gate8 = gate2.reshape(A)
    te = te2.reshape(NT)
    used = used2.reshape(NT)

    BS = 512
    HB = 1024
    sh = pl.pallas_call(
        _shared_body,
        grid=(T // BS, HS // HB),
        in_specs=[
            pl.BlockSpec((BS, D), lambda t, j: (t, 0)),
            pl.BlockSpec((HB, D), lambda t, j: (j, 0)),
            pl.BlockSpec((HB, D), lambda t, j: (j, 0)),
            pl.BlockSpec((D, HB), lambda t, j: (0, j)),
        ],
        out_specs=pl.BlockSpec((BS, D), lambda t, j: (t, 0)),
        out_shape=jax.ShapeDtypeStruct((T, D), jnp.float32),
    )(x2, shared_w1, shared_w2, shared_w3)

    tok_sorted, gate_sorted = pl.kernel(
        _sc_scatter_body,
        out_type=[jax.ShapeDtypeStruct((P,), jnp.int32),
                  jax.ShapeDtypeStruct((P,), jnp.float32)],
        mesh=_SC_MESH,
        scratch_types=[pltpu.VMEM((A,), jnp.int32),
                       pltpu.VMEM((A,), jnp.float32),
                       pltpu.VMEM((P,), jnp.int32),
                       pltpu.VMEM((P,), jnp.float32)],
        compiler_params=pltpu.CompilerParams(needs_layout_passes=False),
    )(pos8, gate8)

    y = pl.pallas_call(
        _gmm_body,
        grid_spec=pltpu.PrefetchScalarGridSpec(
            num_scalar_prefetch=2,
            grid=(NT,),
            in_specs=[
                pl.BlockSpec((TM, 1), lambda i, te, us: (i, 0)),
                pl.BlockSpec((T, D), lambda i, te, us: (0, 0)),
                pl.BlockSpec((1, HR, D), lambda i, te, us: (te[i], 0, 0)),
                pl.BlockSpec((1, HR, D), lambda i, te, us: (te[i], 0, 0)),
                pl.BlockSpec((1, D, HR), lambda i, te, us: (te[i], 0, 0)),
                pl.BlockSpec((TM, 1), lambda i, te, us: (i, 0)),
            ],
            out_specs=pl.BlockSpec((TM, D), lambda i, te, us: (i, 0)),
        ),
        out_shape=jax.ShapeDtypeStruct((P, D), jnp.float32),
    )(te, used, tok_sorted.reshape(P, 1), xb, routed_w1, routed_w2, routed_w3,
      gate_sorted.reshape(P, 1))

    out = pl.kernel(
        _sc_combine_body,
        out_type=jax.ShapeDtypeStruct((T, D), jnp.float32),
        mesh=_SC_MESH,
        scratch_types=[pltpu.VMEM((TCH * K,), jnp.int32),
                       pltpu.VMEM((TCH * K,), jnp.int32),
                       pltpu.VMEM((TCH * K, D), jnp.float32),
                       pltpu.VMEM((TCH * K, D), jnp.float32),
                       pltpu.VMEM((TCH, D), jnp.float32),
                       pltpu.VMEM((TCH, D), jnp.float32),
                       pltpu.SemaphoreType.DMA,
                       pltpu.SemaphoreType.DMA,
                       pltpu.SemaphoreType.DMA,
                       pltpu.SemaphoreType.DMA],
    )(y, pos8, sh)

    return out.reshape(b, s, d)


# speedup vs baseline: 2.1151x; 1.0184x over previous
import jax
import jax.numpy as jnp
from jax import lax
from jax.experimental import pallas as pl
from jax.experimental.pallas import tpu as pltpu
from jax.experimental.pallas import tpu_sc as plsc

T = 4096
D = 1024
E = 16
HS = 4096
HR = 1024
K = 2
A = T * K
TM = 256
NT = A // TM + E
P = NT * TM
NC, NS, L = 2, 16, 16
NW = NC * NS


def _dotT(a, b):
    return jax.lax.dot_general(a, b, (((1,), (1,)), ((), ())),
                               preferred_element_type=jnp.float32)


def _dot_hi(a, b):
    return jax.lax.dot_general(a, b, (((1,), (0,)), ((), ())),
                               preferred_element_type=jnp.float32,
                               precision=jax.lax.Precision.HIGHEST)


def _router_body(x_ref, rw_ref, bias_ref, pos_ref, gate_ref, te_ref, used_ref,
                 xb_ref):
    x = x_ref[...]
    xb_ref[...] = x.astype(jnp.bfloat16)
    scores = jax.nn.sigmoid(_dotT(x, rw_ref[...]))
    sel = scores + bias_ref[...]
    cols = lax.broadcasted_iota(jnp.int32, (T, E), 1)
    m1 = jnp.max(sel, axis=1, keepdims=True)
    i1 = jnp.min(jnp.where(sel == m1, cols, E), axis=1, keepdims=True)
    mask1 = cols == i1
    sel2 = jnp.where(mask1, -jnp.inf, sel)
    m2 = jnp.max(sel2, axis=1, keepdims=True)
    i2 = jnp.min(jnp.where(sel2 == m2, cols, E), axis=1, keepdims=True)
    mask2 = cols == i2
    selk = (mask1 | mask2).astype(jnp.float32)

    BB = 512
    rix = lax.broadcasted_iota(jnp.int32, (BB, BB), 0)
    cix = lax.broadcasted_iota(jnp.int32, (BB, BB), 1)
    tril = (cix < rix).astype(jnp.float32)
    carry = jnp.zeros((1, E), jnp.float32)
    rblocks = []
    for bi in range(T // BB):
        blk = selk[bi * BB:(bi + 1) * BB, :]
        rblocks.append(_dot_hi(tril, blk) + carry)
        carry = carry + jnp.sum(blk, axis=0, keepdims=True)
    ranks = jnp.concatenate(rblocks, axis=0)
    counts = carry
    padded = jnp.ceil(counts / TM) * TM

    eri = lax.broadcasted_iota(jnp.int32, (E, E), 0)
    eci = lax.broadcasted_iota(jnp.int32, (E, E), 1)
    etril = (eri < eci).astype(jnp.float32)
    seg = _dot_hi(padded, etril)

    posd = seg + ranks
    p1 = jnp.sum(jnp.where(mask1, posd, 0.0), axis=1, keepdims=True)
    p2 = jnp.sum(jnp.where(mask2, posd, 0.0), axis=1, keepdims=True)
    g1 = jnp.sum(jnp.where(mask1, scores, 0.0), axis=1, keepdims=True)
    g2 = jnp.sum(jnp.where(mask2, scores, 0.0), axis=1, keepdims=True)
    pos_ref[...] = jnp.concatenate([p1, p2], axis=1).astype(jnp.int32)
    gate_ref[...] = jnp.concatenate([g1, g2], axis=1)

    tstart = (lax.broadcasted_iota(jnp.int32, (NT, 1), 0) * TM).astype(jnp.float32)
    te_ref[...] = (jnp.sum((seg <= tstart).astype(jnp.int32), axis=1,
                           keepdims=True) - 1)
    total = jnp.sum(padded, axis=1, keepdims=True)
    used_ref[...] = (tstart < total).astype(jnp.int32)


def _sc_scatter_body(pos_hbm, gate_hbm, tok_out, gsort_out,
                     posv, gatev, tokbuf, gatebuf):
    wid = lax.axis_index("s") * NC + lax.axis_index("c")

    @pl.when(wid == 0)
    def _():
        pltpu.sync_copy(pos_hbm, posv)
        pltpu.sync_copy(gate_hbm, gatev)

        def zero_body(i, c):
            tokbuf[pl.ds(i * L, L)] = jnp.zeros((L,), jnp.int32)
            gatebuf[pl.ds(i * L, L)] = jnp.zeros((L,), jnp.float32)
            return c
        lax.fori_loop(0, P // L, zero_body, 0)

        iota = lax.iota(jnp.int32, L)

        def scat_body(j, c):
            idx = posv[pl.ds(j * L, L)]
            tok = lax.shift_right_logical(iota + j * L, 1)
            g = gatev[pl.ds(j * L, L)]
            plsc.store_scatter(tokbuf, [idx], tok)
            plsc.store_scatter(gatebuf, [idx], g)
            return c
        lax.fori_loop(0, A // L, scat_body, 0)

        pltpu.sync_copy(tokbuf, tok_out)
        pltpu.sync_copy(gatebuf, gsort_out)


def _gmm_body(te_ref, used_ref, tok_ref, xb_ref, w1_ref, w2_ref, w3_ref,
              g_ref, y_ref):
    i = pl.program_id(0)

    @pl.when(used_ref[i] != 0)
    def _():
        tok = tok_ref[...]
        tcols = lax.broadcasted_iota(jnp.int32, (TM, T), 1)
        onehot = (tcols == tok).astype(jnp.bfloat16)
        xg = jax.lax.dot_general(onehot, xb_ref[...], (((1,), (0,)), ((), ())),
                                 preferred_element_type=jnp.float32)
        h = jax.nn.silu(_dotT(xg, w1_ref[0])) * _dotT(xg, w2_ref[0])
        y_ref[...] = _dotT(h, w3_ref[0]) * g_ref[...]


_BS = 512
_HB = 1024
_NJ = HS // _HB


def _shared_body(x_ref, w1_ref, w2_ref, w3_ref, out_ref, acc_ref):
    j = pl.program_id(0)
    t = pl.program_id(1)
    x = x_ref[...]
    h = jax.nn.silu(_dotT(x, w1_ref[...])) * _dotT(x, w2_ref[...])
    part = _dotT(h, w3_ref[...])
    sl = pl.ds(t * _BS, _BS)

    @pl.when(j == 0)
    def _():
        acc_ref[sl, :] = part

    @pl.when((j != 0) & (j != _NJ - 1))
    def _():
        acc_ref[sl, :] += part

    @pl.when(j == _NJ - 1)
    def _():
        out_ref[...] = acc_ref[sl, :] + part


TPW = T // NW
TCH = 16


def _sc_combine_body(y_hbm, pos_hbm, sh_hbm, out_hbm,
                     idx0, idx1, yb0, yb1, acc0, acc1, g0, g1, s0, s1):
    idxs, ybs, accs = [idx0, idx1], [yb0, yb1], [acc0, acc1]
    gsems, ssems = [g0, g1], [s0, s1]
    wid = lax.axis_index("s") * NC + lax.axis_index("c")
    tbase = wid * TPW
    nch = TPW // TCH
    pend = {}

    def start(c):
        k = c & 1
        t0 = tbase + c * TCH
        pltpu.sync_copy(pos_hbm.at[pl.ds(t0 * K, TCH * K)], idxs[k])
        pend[c] = (pltpu.async_copy(y_hbm.at[idxs[k]], ybs[k], gsems[k]),
                   pltpu.async_copy(sh_hbm.at[pl.ds(t0, TCH)], accs[k],
                                    ssems[k]))

    start(0)
    for c in range(nch):
        k = c & 1
        t0 = tbase + c * TCH
        cp_y, cp_s = pend.pop(c)
        cp_y.wait()
        cp_s.wait()
        if c + 1 < nch:
            start(c + 1)
        acc, yb = accs[k], ybs[k]
        for i in range(TCH):
            def vbody(v, c2, i=i):
                sl = pl.ds(v * L, L)
                acc[i, sl] = acc[i, sl] + yb[2 * i, sl] + yb[2 * i + 1, sl]
                return c2
            lax.fori_loop(0, D // L, vbody, 0)
        pltpu.sync_copy(acc, out_hbm.at[pl.ds(t0, TCH)])


def _sc_mesh():
    return plsc.VectorSubcoreMesh(core_axis_name="c", subcore_axis_name="s",
                                  num_cores=NC, num_subcores=NS)


def kernel(x, shared_w1, shared_w2, shared_w3, routed_w1, routed_w2, routed_w3,
           router_w, expert_bias):
    b, s, d = x.shape
    _SC_MESH = _sc_mesh()
    x2 = x.reshape(T, D)

    pos2, gate2, te2, used2, xb = pl.pallas_call(
        _router_body,
        out_shape=[
            jax.ShapeDtypeStruct((T, K), jnp.int32),
            jax.ShapeDtypeStruct((T, K), jnp.float32),
            jax.ShapeDtypeStruct((NT, 1), jnp.int32),
            jax.ShapeDtypeStruct((NT, 1), jnp.int32),
            jax.ShapeDtypeStruct((T, D), jnp.bfloat16),
        ],
    )(x2, router_w, expert_bias.reshape(1, E))
    pos8 = pos2.reshape(A)
    gate8 = gate2.reshape(A)
    te = te2.reshape(NT)
    used = used2.reshape(NT)

    sh = pl.pallas_call(
        _shared_body,
        grid=(_NJ, T // _BS),
        in_specs=[
            pl.BlockSpec((_BS, D), lambda j, t: (t, 0)),
            pl.BlockSpec((_HB, D), lambda j, t: (j, 0)),
            pl.BlockSpec((_HB, D), lambda j, t: (j, 0)),
            pl.BlockSpec((D, _HB), lambda j, t: (0, j)),
        ],
        out_specs=pl.BlockSpec((_BS, D), lambda j, t: (t, 0)),
        out_shape=jax.ShapeDtypeStruct((T, D), jnp.float32),
        scratch_shapes=[pltpu.VMEM((T, D), jnp.float32)],
    )(x2, shared_w1, shared_w2, shared_w3)

    tok_sorted, gate_sorted = pl.kernel(
        _sc_scatter_body,
        out_type=[jax.ShapeDtypeStruct((P,), jnp.int32),
                  jax.ShapeDtypeStruct((P,), jnp.float32)],
        mesh=_SC_MESH,
        scratch_types=[pltpu.VMEM((A,), jnp.int32),
                       pltpu.VMEM((A,), jnp.float32),
                       pltpu.VMEM((P,), jnp.int32),
                       pltpu.VMEM((P,), jnp.float32)],
        compiler_params=pltpu.CompilerParams(needs_layout_passes=False),
    )(pos8, gate8)

    y = pl.pallas_call(
        _gmm_body,
        grid_spec=pltpu.PrefetchScalarGridSpec(
            num_scalar_prefetch=2,
            grid=(NT,),
            in_specs=[
                pl.BlockSpec((TM, 1), lambda i, te, us: (i, 0)),
                pl.BlockSpec((T, D), lambda i, te, us: (0, 0)),
                pl.BlockSpec((1, HR, D), lambda i, te, us: (te[i], 0, 0)),
                pl.BlockSpec((1, HR, D), lambda i, te, us: (te[i], 0, 0)),
                pl.BlockSpec((1, D, HR), lambda i, te, us: (te[i], 0, 0)),
                pl.BlockSpec((TM, 1), lambda i, te, us: (i, 0)),
            ],
            out_specs=pl.BlockSpec((TM, D), lambda i, te, us: (i, 0)),
        ),
        out_shape=jax.ShapeDtypeStruct((P, D), jnp.float32),
    )(te, used, tok_sorted.reshape(P, 1), xb, routed_w1, routed_w2, routed_w3,
      gate_sorted.reshape(P, 1))

    out = pl.kernel(
        _sc_combine_body,
        out_type=jax.ShapeDtypeStruct((T, D), jnp.float32),
        mesh=_SC_MESH,
        scratch_types=[pltpu.VMEM((TCH * K,), jnp.int32),
                       pltpu.VMEM((TCH * K,), jnp.int32),
                       pltpu.VMEM((TCH * K, D), jnp.float32),
                       pltpu.VMEM((TCH * K, D), jnp.float32),
                       pltpu.VMEM((TCH, D), jnp.float32),
                       pltpu.VMEM((TCH, D), jnp.float32),
                       pltpu.SemaphoreType.DMA,
                       pltpu.SemaphoreType.DMA,
                       pltpu.SemaphoreType.DMA,
                       pltpu.SemaphoreType.DMA],
    )(y, pos8, sh)

    return out.reshape(b, s, d)
